# baseline (speedup 1.0000x reference)
import jax
import jax.numpy as jnp
from jax.experimental import pallas as pl
from jax.experimental.pallas import tpu as pltpu

_K = 50
_NITER = 20
_SPAN = 128.0


def _soft_block(tgt_bf, seq, slen):
    Bb, L, D = seq.shape
    slen = slen.reshape(1, Bb)
    seq_flat = seq.reshape(Bb * L, D)
    seq_bf = seq_flat.astype(jnp.bfloat16)
    r = jax.lax.dot_general(seq_bf, tgt_bf, (((1,), (1,)), ((), ())),
                            preferred_element_type=jnp.float32)
    r3 = r.reshape(Bb, L, Bb)
    bsel = (jax.lax.broadcasted_iota(jnp.int32, (Bb, 1, Bb), 0)
            == jax.lax.broadcasted_iota(jnp.int32, (Bb, 1, Bb), 2))
    scores_t = jnp.sum(jnp.where(bsel, r3, jnp.float32(0.0)), axis=0)
    pos_t = jax.lax.broadcasted_iota(jnp.int32, (L, Bb), 0)
    valid = pos_t >= (L - slen)
    scores_t = jnp.where(valid, scores_t, jnp.float32(-1e9))
    rowmax = jnp.max(scores_t, axis=0, keepdims=True)
    lo = rowmax - jnp.float32(_SPAN)
    hi = rowmax
    kf = jnp.float32(_K)

    def body(_, carry):
        lo, hi = carry
        mid = 0.5 * (lo + hi)
        cnt = jnp.sum((scores_t >= mid).astype(jnp.float32), axis=0,
                      keepdims=True)
        ge = cnt >= kf
        return jnp.where(ge, mid, lo), jnp.where(ge, hi, mid)

    lo, _ = jax.lax.fori_loop(0, _NITER, body, (lo, hi))
    w_t = jnp.where(scores_t >= lo, jnp.exp(scores_t - rowmax),
                    jnp.float32(0.0))
    w_t = jnp.where(slen == 0, (pos_t < _K).astype(jnp.float32), w_t)
    w_t = w_t / jnp.sum(w_t, axis=0, keepdims=True)

    w_b = w_t.T
    bsel2 = (jax.lax.broadcasted_iota(jnp.int32, (Bb, Bb, 1), 0)
             == jax.lax.broadcasted_iota(jnp.int32, (Bb, Bb, 1), 1))
    w2 = jnp.where(bsel2, w_b[:, None, :], jnp.float32(0.0))
    w2 = w2.reshape(Bb, Bb * L)
    return jax.lax.dot_general(w2, seq_flat, (((1,), (0,)), ((), ())),
                               precision=jax.lax.Precision.HIGHEST,
                               preferred_element_type=jnp.float32)


def _fused(tgt_ref, click_ref, clen_ref, exp_ref, elen_ref,
           w1_ref, b1_ref, w2_ref, b2_ref, w3_ref, b3_ref, out_ref):
    tgt_bf = tgt_ref[:].astype(jnp.bfloat16)
    c = _soft_block(tgt_bf, click_ref[:], clen_ref[:])
    e = _soft_block(tgt_bf, exp_ref[:], elen_ref[:])
    h = jnp.concatenate([c, e], axis=1)
    h = jnp.maximum(jnp.dot(h, w1_ref[:], preferred_element_type=jnp.float32)
                    + b1_ref[:], 0.0)
    h = jnp.maximum(jnp.dot(h, w2_ref[:], preferred_element_type=jnp.float32)
                    + b2_ref[:], 0.0)
    h = jnp.maximum(jnp.dot(h, w3_ref[:], preferred_element_type=jnp.float32)
                    + b3_ref[:], 0.0)
    out_ref[:] = h


def kernel(tgt_emb, click_emb, click_len, exposure_emb, exposure_len,
           W1, b1, W2, b2, W3, b3):
    B, L, D = click_emb.shape
    Bb = 64
    grid = (B // Bb,)
    clen = click_len.reshape(B // Bb, 1, Bb)
    elen = exposure_len.reshape(B // Bb, 1, Bb)
    b1r = b1.reshape(1, -1)
    b2r = b2.reshape(1, -1)
    b3r = b3.reshape(1, -1)
    u1, u2, u3 = W1.shape[1], W2.shape[1], W3.shape[1]

    row = lambda i: (i, 0)
    row3 = lambda i: (i, 0, 0)
    col = lambda i: (0, i)
    rep = lambda i: (0, 0)

    out = pl.pallas_call(
        _fused,
        grid=grid,
        in_specs=[
            pl.BlockSpec((Bb, D), row),
            pl.BlockSpec((Bb, L, D), row3),
            pl.BlockSpec((1, 1, Bb), row3),
            pl.BlockSpec((Bb, L, D), row3),
            pl.BlockSpec((1, 1, Bb), row3),
            pl.BlockSpec((2 * D, u1), rep),
            pl.BlockSpec((1, u1), rep),
            pl.BlockSpec((u1, u2), rep),
            pl.BlockSpec((1, u2), rep),
            pl.BlockSpec((u2, u3), rep),
            pl.BlockSpec((1, u3), rep),
        ],
        out_specs=pl.BlockSpec((Bb, u3), row),
        out_shape=jax.ShapeDtypeStruct((B, u3), jnp.float32),
        compiler_params=pltpu.CompilerParams(
            dimension_semantics=("arbitrary",),
        ),
    )(tgt_emb, click_emb, clen, exposure_emb, elen,
      W1, b1r, W2, b2r, W3, b3r)
    return out[:, None, :]


# speedup vs baseline: 1.7726x; 1.7726x over previous
import jax
import jax.numpy as jnp
from jax.experimental import pallas as pl
from jax.experimental.pallas import tpu as pltpu

_K = 50
_NITER = 18
_SPAN = 128.0


def _scores(tgt_bf, seq, slen):
    Bb, L, D = seq.shape
    seq_flat = seq.reshape(Bb * L, D)
    seq_bf = seq_flat.astype(jnp.bfloat16)
    seq_lo = (seq_flat - seq_bf.astype(jnp.float32)).astype(jnp.bfloat16)
    r = jax.lax.dot_general(seq_bf, tgt_bf, (((1,), (1,)), ((), ())),
                            preferred_element_type=jnp.float32)
    r3 = r.reshape(Bb, L, Bb)
    bsel = (jax.lax.broadcasted_iota(jnp.int32, (Bb, 1, Bb), 0)
            == jax.lax.broadcasted_iota(jnp.int32, (Bb, 1, Bb), 2))
    scores_t = jnp.sum(jnp.where(bsel, r3, jnp.float32(0.0)), axis=0)
    pos_t = jax.lax.broadcasted_iota(jnp.int32, (L, Bb), 0)
    valid = pos_t >= (L - slen)
    scores_t = jnp.where(valid, scores_t, jnp.float32(-1e9))
    return scores_t, seq_bf, seq_lo


def _weights(scores_t, slen2):
    L, N = scores_t.shape
    rowmax = jnp.max(scores_t, axis=0, keepdims=True)
    lo = rowmax - jnp.float32(_SPAN)
    hi = rowmax
    kf = jnp.float32(_K)

    def body(_, carry):
        lo, hi = carry
        mid = 0.5 * (lo + hi)
        cnt = jnp.sum((scores_t >= mid).astype(jnp.float32), axis=0,
                      keepdims=True)
        ge = cnt >= kf
        return jnp.where(ge, mid, lo), jnp.where(ge, hi, mid)

    lo, _ = jax.lax.fori_loop(0, _NITER, body, (lo, hi))
    w_t = jnp.where(scores_t >= lo, jnp.exp(scores_t - rowmax),
                    jnp.float32(0.0))
    pos_t = jax.lax.broadcasted_iota(jnp.int32, (L, N), 0)
    w_t = jnp.where(slen2 == 0, (pos_t < _K).astype(jnp.float32), w_t)
    return w_t / jnp.sum(w_t, axis=0, keepdims=True)


def _wsum(w_t, seq_bf, seq_lo):
    L, Bb = w_t.shape
    w_b = w_t.T
    bsel2 = (jax.lax.broadcasted_iota(jnp.int32, (Bb, Bb, 1), 0)
             == jax.lax.broadcasted_iota(jnp.int32, (Bb, Bb, 1), 1))
    w2f = jnp.where(bsel2, w_b[:, None, :], jnp.float32(0.0))
    w2f = w2f.reshape(Bb, Bb * L)
    w2 = w2f.astype(jnp.bfloat16)
    w2_lo = (w2f - w2.astype(jnp.float32)).astype(jnp.bfloat16)
    dn = (((1,), (0,)), ((), ()))
    return (jax.lax.dot_general(w2, seq_bf, dn,
                                preferred_element_type=jnp.float32)
            + jax.lax.dot_general(w2, seq_lo, dn,
                                  preferred_element_type=jnp.float32)
            + jax.lax.dot_general(w2_lo, seq_bf, dn,
                                  preferred_element_type=jnp.float32))


def _fused(tgt_ref, click_ref, clen_ref, exp_ref, elen_ref,
           w1_ref, b1_ref, w2_ref, b2_ref, w3_ref, b3_ref, out_ref):
    Bb = tgt_ref.shape[0]
    tgt_bf = tgt_ref[:].astype(jnp.bfloat16)
    clen = clen_ref[:].reshape(1, Bb)
    elen = elen_ref[:].reshape(1, Bb)
    sc, cbf, clo = _scores(tgt_bf, click_ref[:], clen)
    se, ebf, elo = _scores(tgt_bf, exp_ref[:], elen)
    w_both = _weights(jnp.concatenate([sc, se], axis=1),
                      jnp.concatenate([clen, elen], axis=1))
    c = _wsum(w_both[:, :Bb], cbf, clo)
    e = _wsum(w_both[:, Bb:], ebf, elo)
    h = jnp.concatenate([c, e], axis=1)
    h = jnp.maximum(jnp.dot(h, w1_ref[:], preferred_element_type=jnp.float32)
                    + b1_ref[:], 0.0)
    h = jnp.maximum(jnp.dot(h, w2_ref[:], preferred_element_type=jnp.float32)
                    + b2_ref[:], 0.0)
    h = jnp.maximum(jnp.dot(h, w3_ref[:], preferred_element_type=jnp.float32)
                    + b3_ref[:], 0.0)
    out_ref[:] = h


def kernel(tgt_emb, click_emb, click_len, exposure_emb, exposure_len,
           W1, b1, W2, b2, W3, b3):
    B, L, D = click_emb.shape
    Bb = 64
    grid = (B // Bb,)
    clen = click_len.reshape(B // Bb, 1, Bb)
    elen = exposure_len.reshape(B // Bb, 1, Bb)
    b1r = b1.reshape(1, -1)
    b2r = b2.reshape(1, -1)
    b3r = b3.reshape(1, -1)
    u1, u2, u3 = W1.shape[1], W2.shape[1], W3.shape[1]

    row = lambda i: (i, 0)
    row3 = lambda i: (i, 0, 0)
    rep = lambda i: (0, 0)

    out = pl.pallas_call(
        _fused,
        grid=grid,
        in_specs=[
            pl.BlockSpec((Bb, D), row),
            pl.BlockSpec((Bb, L, D), row3),
            pl.BlockSpec((1, 1, Bb), row3),
            pl.BlockSpec((Bb, L, D), row3),
            pl.BlockSpec((1, 1, Bb), row3),
            pl.BlockSpec((2 * D, u1), rep),
            pl.BlockSpec((1, u1), rep),
            pl.BlockSpec((u1, u2), rep),
            pl.BlockSpec((1, u2), rep),
            pl.BlockSpec((u2, u3), rep),
            pl.BlockSpec((1, u3), rep),
        ],
        out_specs=pl.BlockSpec((Bb, u3), row),
        out_shape=jax.ShapeDtypeStruct((B, u3), jnp.float32),
        compiler_params=pltpu.CompilerParams(
            dimension_semantics=("arbitrary",),
        ),
    )(tgt_emb, click_emb, clen, exposure_emb, elen,
      W1, b1r, W2, b2r, W3, b3r)
    return out[:, None, :]


# speedup vs baseline: 2.5566x; 1.4423x over previous
import jax
import jax.numpy as jnp
from jax.experimental import pallas as pl
from jax.experimental.pallas import tpu as pltpu

_K = 50
_NITER = 18
_SPAN = 128.0
_G = 16


def _scores(tgt_bf, seq_bf3, slen):
    Bb, L, D = seq_bf3.shape
    bsel = (jax.lax.broadcasted_iota(jnp.int32, (_G, 1, _G), 0)
            == jax.lax.broadcasted_iota(jnp.int32, (_G, 1, _G), 2))
    cols = []
    for g in range(0, Bb, _G):
        seq_g = seq_bf3[g:g + _G].reshape(_G * L, D)
        r = jax.lax.dot_general(seq_g, tgt_bf[g:g + _G],
                                (((1,), (1,)), ((), ())),
                                preferred_element_type=jnp.float32)
        r3 = r.reshape(_G, L, _G)
        cols.append(jnp.sum(jnp.where(bsel, r3, jnp.float32(0.0)), axis=0))
    scores_t = jnp.concatenate(cols, axis=1)
    pos_t = jax.lax.broadcasted_iota(jnp.int32, (L, Bb), 0)
    valid = pos_t >= (L - slen)
    return jnp.where(valid, scores_t, jnp.float32(-1e9))


def _weights(scores_t, slen2):
    L, N = scores_t.shape
    rowmax = jnp.max(scores_t, axis=0, keepdims=True)
    lo = rowmax - jnp.float32(_SPAN)
    hi = rowmax
    kf = jnp.float32(_K)

    def body(_, carry):
        lo, hi = carry
        mid = 0.5 * (lo + hi)
        cnt = jnp.sum((scores_t >= mid).astype(jnp.float32), axis=0,
                      keepdims=True)
        ge = cnt >= kf
        return jnp.where(ge, mid, lo), jnp.where(ge, hi, mid)

    lo, _ = jax.lax.fori_loop(0, _NITER, body, (lo, hi))
    w_t = jnp.where(scores_t >= lo, jnp.exp(scores_t - rowmax),
                    jnp.float32(0.0))
    pos_t = jax.lax.broadcasted_iota(jnp.int32, (L, N), 0)
    w_t = jnp.where(slen2 == 0, (pos_t < _K).astype(jnp.float32), w_t)
    return w_t / jnp.sum(w_t, axis=0, keepdims=True)


def _wsum(w_rows, seq_bf3):
    Bb, L, D = seq_bf3.shape
    bsel2 = (jax.lax.broadcasted_iota(jnp.int32, (_G, _G, 1), 0)
             == jax.lax.broadcasted_iota(jnp.int32, (_G, _G, 1), 1))
    dn = (((1,), (0,)), ((), ()))
    outs = []
    for g in range(0, Bb, _G):
        seq_g = seq_bf3[g:g + _G].reshape(_G * L, D)
        w2f = jnp.where(bsel2, w_rows[g:g + _G][:, None, :],
                        jnp.float32(0.0)).reshape(_G, _G * L)
        w2h = w2f.astype(jnp.bfloat16)
        w2l = (w2f - w2h.astype(jnp.float32)).astype(jnp.bfloat16)
        outs.append(
            jax.lax.dot_general(w2h, seq_g, dn,
                                preferred_element_type=jnp.float32)
            + jax.lax.dot_general(w2l, seq_g, dn,
                                  preferred_element_type=jnp.float32))
    return jnp.concatenate(outs, axis=0)


def _fused(tgt_ref, click_ref, clen_ref, exp_ref, elen_ref,
           w1_ref, b1_ref, w2_ref, b2_ref, w3_ref, b3_ref, out_ref):
    Bb = tgt_ref.shape[0]
    tgt_bf = tgt_ref[:].astype(jnp.bfloat16)
    clen = clen_ref[:].reshape(1, Bb)
    elen = elen_ref[:].reshape(1, Bb)
    cbf = click_ref[:].astype(jnp.bfloat16)
    ebf = exp_ref[:].astype(jnp.bfloat16)
    sc = _scores(tgt_bf, cbf, clen)
    se = _scores(tgt_bf, ebf, elen)
    w_both = _weights(jnp.concatenate([sc, se], axis=1),
                      jnp.concatenate([clen, elen], axis=1))
    w_rows = w_both.T
    c = _wsum(w_rows[:Bb], cbf)
    e = _wsum(w_rows[Bb:], ebf)
    h = jnp.concatenate([c, e], axis=1)
    h = jnp.maximum(jnp.dot(h, w1_ref[:], preferred_element_type=jnp.float32)
                    + b1_ref[:], 0.0)
    h = jnp.maximum(jnp.dot(h, w2_ref[:], preferred_element_type=jnp.float32)
                    + b2_ref[:], 0.0)
    h = jnp.maximum(jnp.dot(h, w3_ref[:], preferred_element_type=jnp.float32)
                    + b3_ref[:], 0.0)
    out_ref[:] = h


def kernel(tgt_emb, click_emb, click_len, exposure_emb, exposure_len,
           W1, b1, W2, b2, W3, b3):
    B, L, D = click_emb.shape
    Bb = 64
    grid = (B // Bb,)
    clen = click_len.reshape(B // Bb, 1, Bb)
    elen = exposure_len.reshape(B // Bb, 1, Bb)
    b1r = b1.reshape(1, -1)
    b2r = b2.reshape(1, -1)
    b3r = b3.reshape(1, -1)
    u1, u2, u3 = W1.shape[1], W2.shape[1], W3.shape[1]

    row = lambda i: (i, 0)
    row3 = lambda i: (i, 0, 0)
    rep = lambda i: (0, 0)

    out = pl.pallas_call(
        _fused,
        grid=grid,
        in_specs=[
            pl.BlockSpec((Bb, D), row),
            pl.BlockSpec((Bb, L, D), row3),
            pl.BlockSpec((1, 1, Bb), row3),
            pl.BlockSpec((Bb, L, D), row3),
            pl.BlockSpec((1, 1, Bb), row3),
            pl.BlockSpec((2 * D, u1), rep),
            pl.BlockSpec((1, u1), rep),
            pl.BlockSpec((u1, u2), rep),
            pl.BlockSpec((1, u2), rep),
            pl.BlockSpec((u2, u3), rep),
            pl.BlockSpec((1, u3), rep),
        ],
        out_specs=pl.BlockSpec((Bb, u3), row),
        out_shape=jax.ShapeDtypeStruct((B, u3), jnp.float32),
        compiler_params=pltpu.CompilerParams(
            dimension_semantics=("arbitrary",),
        ),
    )(tgt_emb, click_emb, clen, exposure_emb, elen,
      W1, b1r, W2, b2r, W3, b3r)
    return out[:, None, :]
